# baseline (speedup 1.0000x reference)
import jax
import jax.numpy as jnp
from jax.experimental import pallas as pl
from jax.experimental.pallas import tpu as pltpu

_N = 8192
_R = 64
_H = _N // 2

_PHYS = {0: 3, 1: 4, 2: 5, 3: 6, 4: 7, 5: 8, 6: 9, 7: 10, 8: 11,
         9: 12, 10: 2, 11: 1, 12: 0}
_PERM8 = [0, 4, 2, 6, 1, 5, 3, 7]


def _log2(v):
    return v.bit_length() - 1


def _sort_body(x_ref, o_ref):
    z = x_ref[:]
    ia = jax.lax.broadcasted_iota(jnp.int32, (_H, 128), 0)
    il = jax.lax.broadcasted_iota(jnp.int32, (_H, 128), 1)
    cp = ia & 7
    rev = ((cp & 1) << 2) | (cp & 2) | (cp >> 2)
    ii = (ia >> 3) + jnp.where(il >= _R, 512, 0) + (rev << 10)
    z = jnp.where((ii & 2) == 0, z, -z)
    k = 2
    while k <= _N:
        j = k // 2
        while j >= 1:
            phys = _PHYS[_log2(j)]
            if phys == 12:
                p = pltpu.roll(z, _R, axis=1)
                z = jnp.where((ii & j) == 0, jnp.minimum(z, p),
                              jnp.maximum(z, p))
            elif phys >= 3:
                s = 1 << phys
                z4 = z.reshape(_H // (2 * s), 2, s, 128)
                a = z4[:, 0]
                b = z4[:, 1]
                z = jnp.concatenate(
                    [jnp.minimum(a, b)[:, None], jnp.maximum(a, b)[:, None]],
                    axis=1).reshape(_H, 128)
            else:
                s = 1 << phys
                fwd = pltpu.roll(z, _H - s, axis=0)
                bwd = pltpu.roll(z, s, axis=0)
                z = jnp.where((ii & j) == 0, jnp.minimum(z, fwd),
                              jnp.maximum(z, bwd))
            j //= 2
        if k < _N:
            z = jnp.where(((ii & k) != 0) == ((ii & (2 * k)) != 0), z, -z)
        k *= 2
    o_ref[:] = z


def kernel(x):
    t = x.reshape(_R, 8, 2, 512)[:, _PERM8]
    zt = t.transpose(3, 1, 2, 0).reshape(_H, 128)
    out = pl.pallas_call(
        _sort_body,
        out_shape=jax.ShapeDtypeStruct((_H, 128), jnp.float32),
    )(zt)
    u = out.reshape(512, 8, 2, _R).transpose(3, 1, 2, 0)[:, _PERM8]
    return u.reshape(_R, _N)


# speedup vs baseline: 1.1477x; 1.1477x over previous
import jax
import jax.numpy as jnp
from jax.experimental import pallas as pl
from jax.experimental.pallas import tpu as pltpu

_N = 8192
_R = 64
_H = _N // 2

_PHYS = {0: 3, 1: 4, 2: 5, 3: 6, 4: 7, 5: 8, 6: 9, 7: 10, 8: 11,
         9: 12, 10: 0, 11: 1, 12: 2}


def _log2(v):
    return v.bit_length() - 1


def _sort_body(x_ref, o_ref):
    z = x_ref[:]
    ia = jax.lax.broadcasted_iota(jnp.int32, (_H, 128), 0)
    il = jax.lax.broadcasted_iota(jnp.int32, (_H, 128), 1)
    ii = ((ia & 7) << 10) + (ia >> 3) + jnp.where(il >= _R, 512, 0)
    z = jnp.where((ii & 2) == 0, z, -z)
    k = 2
    while k <= _N:
        j = k // 2
        while j >= 1:
            phys = _PHYS[_log2(j)]
            if phys == 12:
                p = pltpu.roll(z, _R, axis=1)
                z = jnp.where((ii & j) == 0, jnp.minimum(z, p),
                              jnp.maximum(z, p))
            elif phys >= 3:
                s = 1 << phys
                z4 = z.reshape(_H // (2 * s), 2, s, 128)
                a = z4[:, 0]
                b = z4[:, 1]
                z = jnp.concatenate(
                    [jnp.minimum(a, b)[:, None], jnp.maximum(a, b)[:, None]],
                    axis=1).reshape(_H, 128)
            else:
                s = 1 << phys
                fwd = pltpu.roll(z, _H - s, axis=0)
                bwd = pltpu.roll(z, s, axis=0)
                z = jnp.where((ii & j) == 0, jnp.minimum(z, fwd),
                              jnp.maximum(z, bwd))
            j //= 2
        if k < _N:
            z = jnp.where(((ii & k) != 0) == ((ii & (2 * k)) != 0), z, -z)
        k *= 2
    o_ref[:] = z


def kernel(x):
    zt = x.reshape(_R, 8, 2, 512).transpose(3, 1, 2, 0).reshape(_H, 128)
    out = pl.pallas_call(
        _sort_body,
        out_shape=jax.ShapeDtypeStruct((_H, 128), jnp.float32),
    )(zt)
    return out.reshape(512, 8, 2, _R).transpose(3, 1, 2, 0).reshape(_R, _N)


# speedup vs baseline: 1.3640x; 1.1885x over previous
import jax
import jax.numpy as jnp
from jax.experimental import pallas as pl
from jax.experimental.pallas import tpu as pltpu

_N = 8192
_R = 64
_H = _N // 2

_PHYS = {0: 3, 1: 4, 2: 5, 3: 6, 4: 7, 5: 8, 6: 9, 7: 10, 8: 11,
         9: 0, 10: 1, 11: 2, 12: 12}


def _log2(v):
    return v.bit_length() - 1


def _sort_body(x_ref, o_ref):
    z = x_ref[:]
    z = z.reshape(8, 512, 128).transpose(1, 0, 2).reshape(_H, 128)
    ia = jax.lax.broadcasted_iota(jnp.int32, (_H, 128), 0)
    il = jax.lax.broadcasted_iota(jnp.int32, (_H, 128), 1)
    ii = (ia >> 3) + ((ia & 7) << 9) + jnp.where(il >= _R, 4096, 0)
    z = jnp.where((ii & 2) == 0, z, -z)
    k = 2
    while k <= _N:
        j = k // 2
        while j >= 1:
            phys = _PHYS[_log2(j)]
            if phys == 12:
                p = pltpu.roll(z, _R, axis=1)
                z = jnp.where((ii & j) == 0, jnp.minimum(z, p),
                              jnp.maximum(z, p))
            elif phys >= 3:
                s = 1 << phys
                z4 = z.reshape(_H // (2 * s), 2, s, 128)
                a = z4[:, 0]
                b = z4[:, 1]
                z = jnp.concatenate(
                    [jnp.minimum(a, b)[:, None], jnp.maximum(a, b)[:, None]],
                    axis=1).reshape(_H, 128)
            else:
                s = 1 << phys
                fwd = pltpu.roll(z, _H - s, axis=0)
                bwd = pltpu.roll(z, s, axis=0)
                z = jnp.where((ii & j) == 0, jnp.minimum(z, fwd),
                              jnp.maximum(z, bwd))
            j //= 2
        if k < _N:
            z = jnp.where(((ii & k) != 0) == ((ii & (2 * k)) != 0), z, -z)
        k *= 2
    o_ref[:] = z.reshape(512, 8, 128).transpose(1, 0, 2).reshape(_H, 128)


def kernel(x):
    zt = x.reshape(_R, 2, _H).transpose(2, 1, 0).reshape(_H, 128)
    out = pl.pallas_call(
        _sort_body,
        out_shape=jax.ShapeDtypeStruct((_H, 128), jnp.float32),
    )(zt)
    return out.reshape(_H, 2, _R).transpose(2, 1, 0).reshape(_R, _N)


# speedup vs baseline: 1.3949x; 1.0226x over previous
import jax
import jax.numpy as jnp
from jax.experimental import pallas as pl
from jax.experimental.pallas import tpu as pltpu

_N = 8192
_R = 64
_H = _N // 2

_PHYS = {0: 3, 1: 4, 2: 5, 3: 6, 4: 7, 5: 8, 6: 9, 7: 10, 8: 11,
         9: 0, 10: 1, 11: 2, 12: 12}


def _log2(v):
    return v.bit_length() - 1


def _sort_body(x_ref, o_ref):
    z = x_ref[:]
    z = z.reshape(8, 512, 128).transpose(1, 0, 2).reshape(_H, 128)
    ia = jax.lax.broadcasted_iota(jnp.int32, (_H, 128), 0)
    il = jax.lax.broadcasted_iota(jnp.int32, (_H, 128), 1)
    ii = (ia >> 3) + ((ia & 7) << 9) + jnp.where(il >= _R, 4096, 0)
    z = jnp.where((ii & 2) == 0, z, -z)
    k = 2
    while k <= _N:
        j = k // 2
        while j >= 1:
            phys = _PHYS[_log2(j)]
            if phys == 12:
                p = pltpu.roll(z, _R, axis=1)
                z = jnp.where((ii & j) == 0, jnp.minimum(z, p),
                              jnp.maximum(z, p))
            elif phys >= 3:
                s = 1 << phys
                z4 = z.reshape(_H // (2 * s), 2, s, 128)
                a = z4[:, 0]
                b = z4[:, 1]
                z = jnp.concatenate(
                    [jnp.minimum(a, b)[:, None], jnp.maximum(a, b)[:, None]],
                    axis=1).reshape(_H, 128)
            else:
                s = 1 << phys
                z3 = z.reshape(_H // 8, 8, 128)
                fwd = pltpu.roll(z3, 8 - s, axis=1).reshape(_H, 128)
                bwd = pltpu.roll(z3, s, axis=1).reshape(_H, 128)
                z = jnp.where((ii & j) == 0, jnp.minimum(z, fwd),
                              jnp.maximum(z, bwd))
            j //= 2
        if k < _N:
            z = jnp.where(((ii & k) != 0) == ((ii & (2 * k)) != 0), z, -z)
        k *= 2
    o_ref[:] = z.reshape(512, 8, 128).transpose(1, 0, 2).reshape(_H, 128)


def kernel(x):
    zt = x.reshape(_R, 2, _H).transpose(2, 1, 0).reshape(_H, 128)
    out = pl.pallas_call(
        _sort_body,
        out_shape=jax.ShapeDtypeStruct((_H, 128), jnp.float32),
    )(zt)
    return out.reshape(_H, 2, _R).transpose(2, 1, 0).reshape(_R, _N)
